# baseline (speedup 1.0000x reference)
import functools

import jax
import jax.numpy as jnp
from jax import lax
from jax.experimental import pallas as pl
from jax.experimental.pallas import tpu as pltpu
from jax.experimental.pallas import tpu_sc as plsc

_NUM_CORES = 2
_NUM_SUBCORES = 16
_NUM_WORKERS = _NUM_CORES * _NUM_SUBCORES


@functools.cache
def _make_sc_copy(seq_len, hidden, dtype):
    rows_per_w = seq_len // _NUM_WORKERS
    mesh = plsc.VectorSubcoreMesh(core_axis_name="c", subcore_axis_name="s")

    @functools.partial(
        pl.kernel,
        mesh=mesh,
        out_type=jax.ShapeDtypeStruct((seq_len, hidden), dtype),
    )
    def sc_copy(table_hbm, out_hbm):
        wid = lax.axis_index("s") * _NUM_CORES + lax.axis_index("c")
        base = wid * rows_per_w
        pltpu.sync_copy(
            table_hbm.at[pl.ds(base, rows_per_w)],
            out_hbm.at[pl.ds(base, rows_per_w)],
        )

    return sc_copy


def kernel(x, emb_table):
    seq_len = x.shape[1]
    out = _make_sc_copy(seq_len, emb_table.shape[1], emb_table.dtype)(emb_table)
    return out[None]


# speedup vs baseline: 6.5306x; 6.5306x over previous
import functools

import jax
import jax.numpy as jnp
from jax import lax
from jax.experimental import pallas as pl
from jax.experimental.pallas import tpu as pltpu
from jax.experimental.pallas import tpu_sc as plsc

_NUM_CORES = 2
_NUM_SUBCORES = 16
_NUM_WORKERS = _NUM_CORES * _NUM_SUBCORES


@functools.cache
def _make_sc_copy(seq_len, hidden, dtype):
    rows_per_w = seq_len // _NUM_WORKERS
    mesh = plsc.VectorSubcoreMesh(core_axis_name="c", subcore_axis_name="s")

    @functools.partial(
        pl.kernel,
        mesh=mesh,
        out_type=jax.ShapeDtypeStruct((seq_len, hidden), dtype),
        scratch_types=[pltpu.VMEM((rows_per_w, hidden), dtype)],
    )
    def sc_copy(table_hbm, out_hbm, buf):
        wid = lax.axis_index("s") * _NUM_CORES + lax.axis_index("c")
        base = wid * rows_per_w
        pltpu.sync_copy(table_hbm.at[pl.ds(base, rows_per_w)], buf)
        pltpu.sync_copy(buf, out_hbm.at[pl.ds(base, rows_per_w)])

    return sc_copy


def kernel(x, emb_table):
    seq_len = x.shape[1]
    out = _make_sc_copy(seq_len, emb_table.shape[1], emb_table.dtype)(emb_table)
    return out[None]


# speedup vs baseline: 6.5601x; 1.0045x over previous
import functools

import jax
import jax.numpy as jnp
from jax import lax
from jax.experimental import pallas as pl
from jax.experimental.pallas import tpu as pltpu
from jax.experimental.pallas import tpu_sc as plsc

_NUM_CORES = 2
_NUM_SUBCORES = 16
_NUM_WORKERS = _NUM_CORES * _NUM_SUBCORES
_NUM_CHUNKS = 4


@functools.cache
def _make_sc_copy(seq_len, hidden, dtype):
    rows_per_w = seq_len // _NUM_WORKERS
    mesh = plsc.VectorSubcoreMesh(core_axis_name="c", subcore_axis_name="s")

    @functools.partial(
        pl.kernel,
        mesh=mesh,
        out_type=jax.ShapeDtypeStruct((seq_len, hidden), dtype),
        scratch_types=(
            [pltpu.VMEM((rows_per_w // _NUM_CHUNKS, hidden), dtype)] * _NUM_CHUNKS
            + [pltpu.SemaphoreType.DMA] * _NUM_CHUNKS
            + [pltpu.SemaphoreType.DMA]
        ),
    )
    def sc_copy(table_hbm, out_hbm, *scratch):
        bufs = scratch[:_NUM_CHUNKS]
        in_sems = scratch[_NUM_CHUNKS : 2 * _NUM_CHUNKS]
        out_sem = scratch[2 * _NUM_CHUNKS]
        chunk = rows_per_w // _NUM_CHUNKS
        wid = lax.axis_index("s") * _NUM_CORES + lax.axis_index("c")
        base = wid * rows_per_w
        ins = [
            pltpu.make_async_copy(
                table_hbm.at[pl.ds(base + i * chunk, chunk)], bufs[i], in_sems[i]
            )
            for i in range(_NUM_CHUNKS)
        ]
        for c in ins:
            c.start()
        outs = []
        for i in range(_NUM_CHUNKS):
            ins[i].wait()
            c = pltpu.make_async_copy(
                bufs[i], out_hbm.at[pl.ds(base + i * chunk, chunk)], out_sem
            )
            c.start()
            outs.append(c)
        for c in outs:
            c.wait()

    return sc_copy


def kernel(x, emb_table):
    seq_len = x.shape[1]
    out = _make_sc_copy(seq_len, emb_table.shape[1], emb_table.dtype)(emb_table)
    return out[None]
